# baseline (speedup 1.0000x reference)
import functools

import jax
import jax.numpy as jnp
from jax import lax
from jax.experimental import pallas as pl
from jax.experimental.pallas import tpu as pltpu
from jax.experimental.pallas import tpu_sc as plsc

_NC = 2
_NS = 16
_L = 16


def _make_sc_gather(N, V, D, F, chunk):
    NW = _NC * _NS
    rows_per_w = N // NW
    n_chunks = rows_per_w // chunk
    assert rows_per_w % chunk == 0
    assert chunk % _L == 0 and chunk % F == 0 and rows_per_w % F == 0

    mesh = plsc.VectorSubcoreMesh(core_axis_name="c", subcore_axis_name="s")

    @functools.partial(
        pl.kernel,
        mesh=mesh,
        compiler_params=pltpu.CompilerParams(use_tc_tiling_on_sc=False),
        out_type=jax.ShapeDtypeStruct((N, D), jnp.float32),
        scratch_types=[
            pltpu.VMEM((chunk,), jnp.int32),
            pltpu.VMEM((chunk, D), jnp.float32),
            pltpu.SemaphoreType.DMA,
        ],
    )
    def gather_kernel(tab_hbm, idx_hbm, out_hbm, idx_v, rows_v, sem):
        wid = lax.axis_index("s") * _NC + lax.axis_index("c")
        base = wid * rows_per_w

        def chunk_body(g, _):
            cbase = pl.multiple_of(base + g * chunk, chunk)
            pltpu.sync_copy(idx_hbm.at[pl.ds(cbase, chunk)], idx_v)

            def off_body(j, _):
                s = pl.multiple_of(j * _L, _L)
                lane = lax.iota(jnp.int32, _L) + s
                off = (lane % F) * V
                idx_v[pl.ds(s, _L)] = idx_v[pl.ds(s, _L)] + off
                return 0

            lax.fori_loop(0, chunk // _L, off_body, 0)
            pltpu.async_copy(tab_hbm.at[idx_v], rows_v, sem).wait()
            pltpu.sync_copy(rows_v, out_hbm.at[pl.ds(cbase, chunk)])
            return 0

        lax.fori_loop(0, n_chunks, chunk_body, 0)

    return gather_kernel


def kernel(indices, tables):
    B, F = indices.shape
    Ft, V, D = tables.shape
    N = B * F
    flat_tab = tables.reshape(Ft * V, D)
    flat_idx = indices.reshape(N)
    out = _make_sc_gather(N, V, D, F, chunk=1664)(flat_tab, flat_idx)
    return out.reshape(B, F, D)


# speedup vs baseline: 3.5545x; 3.5545x over previous
import functools

import jax
import jax.numpy as jnp
from jax import lax
from jax.experimental import pallas as pl
from jax.experimental.pallas import tpu as pltpu
from jax.experimental.pallas import tpu_sc as plsc

_NC = 2
_NS = 16
_L = 16


def _make_sc_lookup(F, D, V, B, bq):
    NW = _NC * _NS
    n_planes = F * D
    planes_per_w = n_planes // NW
    assert n_planes % NW == 0 and B % bq == 0 and bq % _L == 0

    mesh = plsc.VectorSubcoreMesh(core_axis_name="c", subcore_axis_name="s")

    @functools.partial(
        pl.kernel,
        mesh=mesh,
        compiler_params=pltpu.CompilerParams(needs_layout_passes=False),
        out_type=jax.ShapeDtypeStruct((F, D, B), jnp.float32),
        scratch_types=[
            pltpu.VMEM((V,), jnp.float32),
            pltpu.VMEM((B,), jnp.int32),
            pltpu.VMEM((bq,), jnp.float32),
            pltpu.SemaphoreType.DMA,
        ],
    )
    def lookup_kernel(tab_hbm, idx_hbm, out_hbm, plane_v, idx_v, outq_v, sem):
        wid = lax.axis_index("s") * _NC + lax.axis_index("c")
        p0 = wid * planes_per_w

        def plane_body(i, prev_f):
            p = p0 + i
            f = p // D
            d = p % D

            @pl.when(f != prev_f)
            def _():
                pltpu.sync_copy(idx_hbm.at[f], idx_v)

            pltpu.sync_copy(tab_hbm.at[f, d], plane_v)

            def quarter_body(q, _):
                def group_body(j, _):
                    s = pl.multiple_of(j * _L, _L)
                    iv = idx_v[pl.ds(q * bq + s, _L)]
                    outq_v[pl.ds(s, _L)] = plsc.load_gather(plane_v, [iv])
                    return 0

                lax.fori_loop(0, bq // _L, group_body, 0)
                pltpu.sync_copy(outq_v, out_hbm.at[f, d, pl.ds(q * bq, bq)])
                return 0

            lax.fori_loop(0, B // bq, quarter_body, 0)
            return f

        lax.fori_loop(0, planes_per_w, plane_body, jnp.int32(-1))

    return lookup_kernel


def kernel(indices, tables):
    B, F = indices.shape
    Ft, V, D = tables.shape
    tab_t = jnp.transpose(tables, (0, 2, 1))
    idx_t = jnp.transpose(indices, (1, 0))
    out_fdb = _make_sc_lookup(F, D, V, B, bq=4096)(tab_t, idx_t)
    return jnp.transpose(out_fdb, (2, 0, 1))


# speedup vs baseline: 3.6323x; 1.0219x over previous
import functools

import jax
import jax.numpy as jnp
from jax import lax
from jax.experimental import pallas as pl
from jax.experimental.pallas import tpu as pltpu
from jax.experimental.pallas import tpu_sc as plsc

_NC = 2
_NS = 16
_L = 16
_UNROLL = 8


def _make_sc_lookup(F, D, V, B, bq):
    NW = _NC * _NS
    n_planes = F * D
    planes_per_w = n_planes // NW
    assert n_planes % NW == 0 and B % bq == 0 and bq % _L == 0

    mesh = plsc.VectorSubcoreMesh(core_axis_name="c", subcore_axis_name="s")

    @functools.partial(
        pl.kernel,
        mesh=mesh,
        compiler_params=pltpu.CompilerParams(needs_layout_passes=False),
        out_type=jax.ShapeDtypeStruct((F, D, B), jnp.float32),
        scratch_types=[
            pltpu.VMEM((V,), jnp.float32),
            pltpu.VMEM((B,), jnp.int32),
            pltpu.VMEM((bq,), jnp.float32),
            pltpu.SemaphoreType.DMA,
        ],
    )
    def lookup_kernel(tab_hbm, idx_hbm, out_hbm, plane_v, idx_v, outq_v, sem):
        wid = lax.axis_index("s") * _NC + lax.axis_index("c")
        p0 = wid * planes_per_w

        def plane_body(i, prev_f):
            p = p0 + i
            f = p // D
            d = p % D

            @pl.when(f != prev_f)
            def _():
                pltpu.sync_copy(idx_hbm.at[f], idx_v)

            pltpu.sync_copy(tab_hbm.at[f, d], plane_v)

            def quarter_body(q, _):
                def group_body(j, _):
                    for u in range(_UNROLL):
                        s = pl.multiple_of(j * _L * _UNROLL + u * _L, _L)
                        iv = idx_v[pl.ds(q * bq + s, _L)]
                        outq_v[pl.ds(s, _L)] = plsc.load_gather(plane_v, [iv])
                    return 0

                lax.fori_loop(0, bq // (_L * _UNROLL), group_body, 0)
                pltpu.sync_copy(outq_v, out_hbm.at[f, d, pl.ds(q * bq, bq)])
                return 0

            lax.fori_loop(0, B // bq, quarter_body, 0)
            return f

        lax.fori_loop(0, planes_per_w, plane_body, jnp.int32(-1))

    return lookup_kernel


def kernel(indices, tables):
    B, F = indices.shape
    Ft, V, D = tables.shape
    tab_t = jnp.transpose(tables, (0, 2, 1))
    idx_t = jnp.transpose(indices, (1, 0))
    out_fdb = _make_sc_lookup(F, D, V, B, bq=4096)(tab_t, idx_t)
    return jnp.transpose(out_fdb, (2, 0, 1))
